# baseline (speedup 1.0000x reference)
import functools

import jax
import jax.numpy as jnp
from jax import lax
from jax.experimental import pallas as pl
from jax.experimental.pallas import tpu as pltpu
from jax.experimental.pallas import tpu_sc as plsc

N_NODES = 10000
N_EDGES = 160000
H = 256
ED = 4
HH = 128
ROWS_PER_TILE = 632
N_PAD = 16 * ROWS_PER_TILE
E_PER_TILE = 10240
E_PAD = 16 * E_PER_TILE
CHUNK = 128
SUPER = 1024
N_SUPER = E_PER_TILE // SUPER
IDX_BLOCKS = E_PAD // SUPER

_F32 = jnp.float32


def _fold_body(epw2f, epb2f, mmw1f, mmb1f, epw2b, epb2b, mmw1b, mmb1b,
               wf, cf, wb, cb):
    wf[...] = jnp.dot(epw2f[...], mmw1f[...], preferred_element_type=_F32)
    cf[...] = jnp.dot(epb2f[...], mmw1f[...], preferred_element_type=_F32) + mmb1f[...]
    wb[...] = jnp.dot(epw2b[...], mmw1b[...], preferred_element_type=_F32)
    cb[...] = jnp.dot(epb2b[...], mmw1b[...], preferred_element_type=_F32) + mmb1b[...]


def _prep_body(x, w1f, w1b, xwf, xwb):
    xf = jnp.dot(x[...], w1f[...], preferred_element_type=_F32)
    xwf[0] = xf[:, :HH]
    xwf[1] = xf[:, HH:]
    xb = jnp.dot(x[...], w1b[...], preferred_element_type=_F32)
    xwb[0] = xb[:, :HH]
    xwb[1] = xb[:, HH:]


def _gmat_body(eat, w1, b1, wfold, cfold, g):
    hid = jax.nn.relu(
        lax.dot_general(eat[...], w1[...], (((0,), (0,)), ((), ())),
                        preferred_element_type=_F32) + b1[...])
    gv = jnp.dot(hid, wfold[...], preferred_element_type=_F32) + cfold[...]
    g[0] = gv[:, :HH]
    g[1] = gv[:, HH:]


def _final_body(sfl, sfh, cntf, sbl, sbh, cntb, w2f, b2f, w2b, b2b, al, out):
    hf = (jnp.dot(sfl[...], w2f[:HH, :], preferred_element_type=_F32)
          + jnp.dot(sfh[...], w2f[HH:, :], preferred_element_type=_F32)
          + cntf[:, 0:1] * b2f[...])
    hb = (jnp.dot(sbl[...], w2b[:HH, :], preferred_element_type=_F32)
          + jnp.dot(sbh[...], w2b[HH:, :], preferred_element_type=_F32)
          + cntb[:, 0:1] * b2b[...])
    a = jax.nn.sigmoid(al[0, 0])
    out[...] = a * hf + (1.0 - a) * hb


def _fold_call(epw2f, epb2f, mmw1f, mmb1f, epw2b, epb2b, mmw1b, mmb1b):
    return pl.pallas_call(
        _fold_body,
        out_shape=[jax.ShapeDtypeStruct((H, H), _F32),
                   jax.ShapeDtypeStruct((1, H), _F32),
                   jax.ShapeDtypeStruct((H, H), _F32),
                   jax.ShapeDtypeStruct((1, H), _F32)],
    )(epw2f, epb2f, mmw1f, mmb1f, epw2b, epb2b, mmw1b, mmb1b)


def _prep_call(x, w1f, w1b):
    tn = 1000
    grid = N_NODES // tn
    return pl.pallas_call(
        _prep_body,
        grid=(grid,),
        in_specs=[pl.BlockSpec((tn, H), lambda i: (i, 0)),
                  pl.BlockSpec((H, H), lambda i: (0, 0)),
                  pl.BlockSpec((H, H), lambda i: (0, 0))],
        out_specs=[pl.BlockSpec((2, tn, HH), lambda i: (0, i, 0)),
                   pl.BlockSpec((2, tn, HH), lambda i: (0, i, 0))],
        out_shape=[jax.ShapeDtypeStruct((2, N_NODES, HH), _F32),
                   jax.ShapeDtypeStruct((2, N_NODES, HH), _F32)],
    )(x, w1f, w1b)


def _gmat_call(eat, w1, b1, wfold, cfold):
    te = 2048
    grid = E_PAD // te
    return pl.pallas_call(
        _gmat_body,
        grid=(grid,),
        in_specs=[pl.BlockSpec((ED, te), lambda i: (0, i)),
                  pl.BlockSpec((ED, H), lambda i: (0, 0)),
                  pl.BlockSpec((1, H), lambda i: (0, 0)),
                  pl.BlockSpec((H, H), lambda i: (0, 0)),
                  pl.BlockSpec((1, H), lambda i: (0, 0))],
        out_specs=pl.BlockSpec((2, te, HH), lambda i: (0, i, 0)),
        out_shape=jax.ShapeDtypeStruct((2, E_PAD, HH), _F32),
    )(eat, w1, b1, wfold, cfold)


def _final_call(sfl, sfh, cntf, sbl, sbh, cntb, w2f, b2f, w2b, b2b, al):
    tn = 1000
    grid = N_NODES // tn
    return pl.pallas_call(
        _final_body,
        grid=(grid,),
        in_specs=[pl.BlockSpec((tn, HH), lambda i: (i, 0)),
                  pl.BlockSpec((tn, HH), lambda i: (i, 0)),
                  pl.BlockSpec((tn, 16), lambda i: (i, 0)),
                  pl.BlockSpec((tn, HH), lambda i: (i, 0)),
                  pl.BlockSpec((tn, HH), lambda i: (i, 0)),
                  pl.BlockSpec((tn, 16), lambda i: (i, 0)),
                  pl.BlockSpec((H, H), lambda i: (0, 0)),
                  pl.BlockSpec((1, H), lambda i: (0, 0)),
                  pl.BlockSpec((H, H), lambda i: (0, 0)),
                  pl.BlockSpec((1, H), lambda i: (0, 0)),
                  pl.BlockSpec((1, 1), lambda i: (0, 0))],
        out_specs=pl.BlockSpec((tn, H), lambda i: (i, 0)),
        out_shape=jax.ShapeDtypeStruct((N_NODES, H), _F32),
    )(sfl, sfh, cntf, sbl, sbh, cntb, w2f, b2f, w2b, b2b, al)


def _sc_segment_kernel(gidx, sidx, xw, gmat, zd, s_out,
                       data_sh, gi2, si2, buf0, buf1, sem_g, sem_l, sem_s):
    c = lax.axis_index("c")
    s = lax.axis_index("s")
    rbase = s * ROWS_PER_TILE
    pltpu.sync_copy(zd.at[pl.ds(rbase, ROWS_PER_TILE)],
                    data_sh.at[pl.ds(rbase, ROWS_PER_TILE)])
    plsc.subcore_barrier()

    goff = c * E_PAD
    xoff = c * N_NODES
    n_sub = SUPER // CHUNK
    bufs = (buf0, buf1)

    def super_chunk(m, carry):
        q = s * N_SUPER + m
        pltpu.sync_copy(gidx.at[q], gi2)
        pltpu.sync_copy(sidx.at[q], si2)
        for r in range(8):
            for j in range(8):
                sl = pl.ds(j * 16, 16)
                gi2[r, sl] = gi2[r, sl] + xoff
        sbase = s * E_PER_TILE + m * SUPER
        ld = pltpu.async_copy(gmat.at[pl.ds(goff + sbase, CHUNK)],
                              bufs[0], sem_l)
        ld.wait()
        gd = pltpu.async_copy(xw.at[gi2.at[0]], bufs[0], sem_g, add=True)
        ld = pltpu.async_copy(gmat.at[pl.ds(goff + sbase + CHUNK, CHUNK)],
                              bufs[1], sem_l)
        for t in range(n_sub):
            buf = bufs[t % 2]
            gd.wait()
            if t + 1 < n_sub:
                ld.wait()
                gd = pltpu.async_copy(xw.at[gi2.at[t + 1]],
                                      bufs[(t + 1) % 2], sem_g, add=True)

            def edge(e4, ecarry):
                e = e4 * 4
                for d in range(4):
                    for j in range(8):
                        sl = pl.ds(j * 16, 16)
                        buf[e + d, sl] = jnp.maximum(buf[e + d, sl], 0.0)
                return ecarry

            lax.fori_loop(0, CHUNK // 4, edge, 0)
            sc = pltpu.async_copy(buf, data_sh.at[si2.at[t]], sem_s, add=True)
            sc.wait()
            if t + 2 < n_sub:
                ld = pltpu.async_copy(
                    gmat.at[pl.ds(goff + sbase + (t + 2) * CHUNK, CHUNK)],
                    buf, sem_l)
        return carry

    lax.fori_loop(0, N_SUPER, super_chunk, 0)
    plsc.subcore_barrier()
    pltpu.sync_copy(data_sh.at[pl.ds(rbase, ROWS_PER_TILE)],
                    s_out.at[pl.ds(c * N_PAD + rbase, ROWS_PER_TILE)])


def _sc_call(gidx, sidx, xw, gmat, zd):
    mesh = plsc.VectorSubcoreMesh(core_axis_name="c", subcore_axis_name="s")
    fn = pl.kernel(
        _sc_segment_kernel,
        out_type=jax.ShapeDtypeStruct((2 * N_PAD, HH), _F32),
        mesh=mesh,
        scratch_types=[
            pltpu.VMEM_SHARED((N_PAD, HH), _F32),
            pltpu.VMEM((8, 128), jnp.int32),
            pltpu.VMEM((8, 128), jnp.int32),
            pltpu.VMEM((CHUNK, 128), _F32),
            pltpu.VMEM((CHUNK, 128), _F32),
            pltpu.SemaphoreType.DMA,
            pltpu.SemaphoreType.DMA,
            pltpu.SemaphoreType.DMA,
        ],
    )
    return fn(gidx, sidx, xw, gmat, zd)


def _sc_count_kernel(sidx_f, sidx_b, zc, oneh, cnt_f, cnt_b,
                     cnt_sh, si2, ones_v):
    c = lax.axis_index("c")
    s = lax.axis_index("s")
    rbase = s * ROWS_PER_TILE
    pltpu.sync_copy(zc.at[pl.ds(rbase, ROWS_PER_TILE)],
                    cnt_sh.at[pl.ds(rbase, ROWS_PER_TILE)])
    pltpu.sync_copy(oneh, ones_v)
    plsc.subcore_barrier()

    def body(sidx, cnt_out):
        def super_chunk(m, carry):
            q = s * N_SUPER + m
            pltpu.sync_copy(sidx.at[q], si2)
            for t in range(SUPER // CHUNK):
                pltpu.sync_copy(ones_v, cnt_sh.at[si2.at[t]], add=True)
            return carry

        lax.fori_loop(0, N_SUPER, super_chunk, 0)
        plsc.subcore_barrier()
        pltpu.sync_copy(cnt_sh.at[pl.ds(rbase, ROWS_PER_TILE)],
                        cnt_out.at[pl.ds(rbase, ROWS_PER_TILE)])

    @pl.when(c == 0)
    def _():
        body(sidx_f, cnt_f)

    @pl.when(c == 1)
    def _():
        body(sidx_b, cnt_b)


def _sc_count_call(sidx_f, sidx_b, zc, oneh):
    mesh = plsc.VectorSubcoreMesh(core_axis_name="c", subcore_axis_name="s")
    fn = pl.kernel(
        _sc_count_kernel,
        out_type=(jax.ShapeDtypeStruct((N_PAD, 16), _F32),
                  jax.ShapeDtypeStruct((N_PAD, 16), _F32)),
        mesh=mesh,
        scratch_types=[
            pltpu.VMEM_SHARED((N_PAD, 16), _F32),
            pltpu.VMEM((8, 128), jnp.int32),
            pltpu.VMEM((128, 16), _F32),
        ],
    )
    return fn(sidx_f, sidx_b, zc, oneh)


def kernel(x, edge_index, edge_attr,
           cf_ep_w1, cf_ep_b1, cf_ep_w2, cf_ep_b2,
           cf_mm_w1, cf_mm_b1, cf_mm_w2, cf_mm_b2,
           cb_ep_w1, cb_ep_b1, cb_ep_w2, cb_ep_b2,
           cb_mm_w1, cb_mm_b1, cb_mm_w2, cb_mm_b2, alpha):
    src = edge_index[0]
    dst = edge_index[1]
    pad_e = E_PAD - N_EDGES
    zpad = jnp.zeros((pad_e,), jnp.int32)
    dump = jnp.full((pad_e,), N_NODES, jnp.int32)
    gidx_f = jnp.concatenate([src, zpad]).reshape(IDX_BLOCKS, 8, 128)
    sidx_f = jnp.concatenate([dst, dump]).reshape(IDX_BLOCKS, 8, 128)
    gidx_b = jnp.concatenate([dst, zpad]).reshape(IDX_BLOCKS, 8, 128)
    sidx_b = jnp.concatenate([src, dump]).reshape(IDX_BLOCKS, 8, 128)
    eat = jnp.concatenate([edge_attr, jnp.zeros((pad_e, ED), _F32)]).T

    wf, cfold_f, wb, cfold_b = _fold_call(
        cf_ep_w2, cf_ep_b2.reshape(1, H), cf_mm_w1, cf_mm_b1.reshape(1, H),
        cb_ep_w2, cb_ep_b2.reshape(1, H), cb_mm_w1, cb_mm_b1.reshape(1, H))
    xwf, xwb = _prep_call(x, cf_mm_w1, cb_mm_w1)
    xwf = xwf.reshape(2 * N_NODES, HH)
    xwb = xwb.reshape(2 * N_NODES, HH)
    gf = _gmat_call(eat, cf_ep_w1, cf_ep_b1.reshape(1, H), wf, cfold_f)
    gb = _gmat_call(eat, cb_ep_w1, cb_ep_b1.reshape(1, H), wb, cfold_b)
    gf = gf.reshape(2 * E_PAD, HH)
    gb = gb.reshape(2 * E_PAD, HH)

    zd = jnp.zeros((N_PAD, HH), _F32)
    zc = jnp.zeros((N_PAD, 16), _F32)
    oneh = jnp.zeros((128, 16), _F32).at[:, 0].set(1.0)

    sf = _sc_call(gidx_f, sidx_f, xwf, gf, zd)
    sb = _sc_call(gidx_b, sidx_b, xwb, gb, zd)
    cntf, cntb = _sc_count_call(sidx_f, sidx_b, zc, oneh)

    return _final_call(
        sf[0:N_NODES], sf[N_PAD:N_PAD + N_NODES], cntf[0:N_NODES],
        sb[0:N_NODES], sb[N_PAD:N_PAD + N_NODES], cntb[0:N_NODES],
        cf_mm_w2, cf_mm_b2.reshape(1, H), cb_mm_w2, cb_mm_b2.reshape(1, H),
        alpha.reshape(1, 1))


# speedup vs baseline: 1.0105x; 1.0105x over previous
import functools

import jax
import jax.numpy as jnp
from jax import lax
from jax.experimental import pallas as pl
from jax.experimental.pallas import tpu as pltpu
from jax.experimental.pallas import tpu_sc as plsc

N_NODES = 10000
N_EDGES = 160000
H = 256
ED = 4
HH = 128
ROWS_PER_TILE = 632
N_PAD = 16 * ROWS_PER_TILE
E_PER_TILE = 10240
E_PAD = 16 * E_PER_TILE
CHUNK = 64
SUPER = 1024
N_SUPER = E_PER_TILE // SUPER
IDX_BLOCKS = E_PAD // SUPER

_F32 = jnp.float32


def _fold_body(epw2f, epb2f, mmw1f, mmb1f, epw2b, epb2b, mmw1b, mmb1b,
               wf, cf, wb, cb):
    wf[...] = jnp.dot(epw2f[...], mmw1f[...], preferred_element_type=_F32)
    cf[...] = jnp.dot(epb2f[...], mmw1f[...], preferred_element_type=_F32) + mmb1f[...]
    wb[...] = jnp.dot(epw2b[...], mmw1b[...], preferred_element_type=_F32)
    cb[...] = jnp.dot(epb2b[...], mmw1b[...], preferred_element_type=_F32) + mmb1b[...]


def _prep_body(x, w1f, w1b, xwf, xwb):
    xf = jnp.dot(x[...], w1f[...], preferred_element_type=_F32)
    xwf[0] = xf[:, :HH]
    xwf[1] = xf[:, HH:]
    xb = jnp.dot(x[...], w1b[...], preferred_element_type=_F32)
    xwb[0] = xb[:, :HH]
    xwb[1] = xb[:, HH:]


def _gmat_body(eat, w1, b1, wfold, cfold, g):
    hid = jax.nn.relu(
        lax.dot_general(eat[...], w1[...], (((0,), (0,)), ((), ())),
                        preferred_element_type=_F32) + b1[...])
    gv = jnp.dot(hid, wfold[...], preferred_element_type=_F32) + cfold[...]
    g[0] = gv[:, :HH]
    g[1] = gv[:, HH:]


def _final_body(sfl, sfh, cntf, sbl, sbh, cntb, w2f, b2f, w2b, b2b, al, out):
    hf = (jnp.dot(sfl[...], w2f[:HH, :], preferred_element_type=_F32)
          + jnp.dot(sfh[...], w2f[HH:, :], preferred_element_type=_F32)
          + cntf[:, 0:1] * b2f[...])
    hb = (jnp.dot(sbl[...], w2b[:HH, :], preferred_element_type=_F32)
          + jnp.dot(sbh[...], w2b[HH:, :], preferred_element_type=_F32)
          + cntb[:, 0:1] * b2b[...])
    a = jax.nn.sigmoid(al[0, 0])
    out[...] = a * hf + (1.0 - a) * hb


def _fold_call(epw2f, epb2f, mmw1f, mmb1f, epw2b, epb2b, mmw1b, mmb1b):
    return pl.pallas_call(
        _fold_body,
        out_shape=[jax.ShapeDtypeStruct((H, H), _F32),
                   jax.ShapeDtypeStruct((1, H), _F32),
                   jax.ShapeDtypeStruct((H, H), _F32),
                   jax.ShapeDtypeStruct((1, H), _F32)],
    )(epw2f, epb2f, mmw1f, mmb1f, epw2b, epb2b, mmw1b, mmb1b)


def _prep_call(x, w1f, w1b):
    tn = 1000
    grid = N_NODES // tn
    return pl.pallas_call(
        _prep_body,
        grid=(grid,),
        in_specs=[pl.BlockSpec((tn, H), lambda i: (i, 0)),
                  pl.BlockSpec((H, H), lambda i: (0, 0)),
                  pl.BlockSpec((H, H), lambda i: (0, 0))],
        out_specs=[pl.BlockSpec((2, tn, HH), lambda i: (0, i, 0)),
                   pl.BlockSpec((2, tn, HH), lambda i: (0, i, 0))],
        out_shape=[jax.ShapeDtypeStruct((2, N_NODES, HH), _F32),
                   jax.ShapeDtypeStruct((2, N_NODES, HH), _F32)],
    )(x, w1f, w1b)


def _gmat_call(eat, w1, b1, wfold, cfold):
    te = 2048
    grid = E_PAD // te
    return pl.pallas_call(
        _gmat_body,
        grid=(grid,),
        in_specs=[pl.BlockSpec((ED, te), lambda i: (0, i)),
                  pl.BlockSpec((ED, H), lambda i: (0, 0)),
                  pl.BlockSpec((1, H), lambda i: (0, 0)),
                  pl.BlockSpec((H, H), lambda i: (0, 0)),
                  pl.BlockSpec((1, H), lambda i: (0, 0))],
        out_specs=pl.BlockSpec((2, te, HH), lambda i: (0, i, 0)),
        out_shape=jax.ShapeDtypeStruct((2, E_PAD, HH), _F32),
    )(eat, w1, b1, wfold, cfold)


def _final_call(sfl, sfh, cntf, sbl, sbh, cntb, w2f, b2f, w2b, b2b, al):
    tn = 1000
    grid = N_NODES // tn
    return pl.pallas_call(
        _final_body,
        grid=(grid,),
        in_specs=[pl.BlockSpec((tn, HH), lambda i: (i, 0)),
                  pl.BlockSpec((tn, HH), lambda i: (i, 0)),
                  pl.BlockSpec((tn, 16), lambda i: (i, 0)),
                  pl.BlockSpec((tn, HH), lambda i: (i, 0)),
                  pl.BlockSpec((tn, HH), lambda i: (i, 0)),
                  pl.BlockSpec((tn, 16), lambda i: (i, 0)),
                  pl.BlockSpec((H, H), lambda i: (0, 0)),
                  pl.BlockSpec((1, H), lambda i: (0, 0)),
                  pl.BlockSpec((H, H), lambda i: (0, 0)),
                  pl.BlockSpec((1, H), lambda i: (0, 0)),
                  pl.BlockSpec((1, 1), lambda i: (0, 0))],
        out_specs=pl.BlockSpec((tn, H), lambda i: (i, 0)),
        out_shape=jax.ShapeDtypeStruct((N_NODES, H), _F32),
    )(sfl, sfh, cntf, sbl, sbh, cntb, w2f, b2f, w2b, b2b, al)


def _sc_segment_kernel(gidx, sidx, xw, gmat, zd, s_out,
                       data_sh, gi2, si2, buf0, buf1, buf2, buf3,
                       sem_g, sem_b0, sem_b1, sem_b2, sem_b3):
    c = lax.axis_index("c")
    s = lax.axis_index("s")
    bufs = (buf0, buf1, buf2, buf3)
    bsem = (sem_b0, sem_b1, sem_b2, sem_b3)
    rbase = s * ROWS_PER_TILE
    pltpu.sync_copy(zd.at[pl.ds(rbase, ROWS_PER_TILE)],
                    data_sh.at[pl.ds(rbase, ROWS_PER_TILE)])
    plsc.subcore_barrier()

    goff = c * E_PAD
    xoff = c * N_NODES
    n_sub = SUPER // CHUNK

    def super_chunk(m, carry):
        q = s * N_SUPER + m
        pltpu.sync_copy(gidx.at[q], gi2)
        pltpu.sync_copy(sidx.at[q], si2)
        for r in range(8):
            for j in range(8):
                sl = pl.ds(j * 16, 16)
                gi2[r, sl] = gi2[r, sl] + xoff
        sbase = s * E_PER_TILE + m * SUPER
        ld = [None] * n_sub
        sc = [None] * n_sub
        for k in range(3):
            ld[k] = pltpu.async_copy(
                gmat.at[pl.ds(goff + sbase + k * CHUNK, CHUNK)],
                bufs[k], bsem[k])
        ld[0].wait()
        gd = pltpu.async_copy(
            xw.at[gi2.at[0, pl.ds(0, CHUNK)]], bufs[0], sem_g, add=True)
        for t in range(n_sub):
            buf = bufs[t % 4]
            gd.wait()
            if t + 1 < n_sub:
                u = t + 1
                ld[u].wait()
                gd = pltpu.async_copy(
                    xw.at[gi2.at[u // 2, pl.ds((u % 2) * CHUNK, CHUNK)]],
                    bufs[u % 4], sem_g, add=True)

            def edge(e, ecarry):
                for j in range(8):
                    sl = pl.ds(j * 16, 16)
                    buf[e, sl] = jnp.maximum(buf[e, sl], 0.0)
                return ecarry

            lax.fori_loop(0, CHUNK, edge, 0)
            sc[t] = pltpu.async_copy(
                buf, data_sh.at[si2.at[t // 2, pl.ds((t % 2) * CHUNK, CHUNK)]],
                bsem[t % 4], add=True)
            if t + 3 < n_sub:
                if t - 1 >= 0:
                    sc[t - 1].wait()
                ld[t + 3] = pltpu.async_copy(
                    gmat.at[pl.ds(goff + sbase + (t + 3) * CHUNK, CHUNK)],
                    bufs[(t + 3) % 4], bsem[(t + 3) % 4])
        for t in range(n_sub - 4, n_sub):
            sc[t].wait()
        return carry

    lax.fori_loop(0, N_SUPER, super_chunk, 0)
    plsc.subcore_barrier()
    pltpu.sync_copy(data_sh.at[pl.ds(rbase, ROWS_PER_TILE)],
                    s_out.at[pl.ds(c * N_PAD + rbase, ROWS_PER_TILE)])


def _sc_call(gidx, sidx, xw, gmat, zd):
    mesh = plsc.VectorSubcoreMesh(core_axis_name="c", subcore_axis_name="s")
    fn = pl.kernel(
        _sc_segment_kernel,
        out_type=jax.ShapeDtypeStruct((2 * N_PAD, HH), _F32),
        mesh=mesh,
        scratch_types=[
            pltpu.VMEM_SHARED((N_PAD, HH), _F32),
            pltpu.VMEM((8, 128), jnp.int32),
            pltpu.VMEM((8, 128), jnp.int32),
            pltpu.VMEM((CHUNK, 128), _F32),
            pltpu.VMEM((CHUNK, 128), _F32),
            pltpu.VMEM((CHUNK, 128), _F32),
            pltpu.VMEM((CHUNK, 128), _F32),
            pltpu.SemaphoreType.DMA,
            pltpu.SemaphoreType.DMA,
            pltpu.SemaphoreType.DMA,
            pltpu.SemaphoreType.DMA,
            pltpu.SemaphoreType.DMA,
        ],
    )
    return fn(gidx, sidx, xw, gmat, zd)


def _sc_count_kernel(sidx_f, sidx_b, zc, oneh, cnt_f, cnt_b,
                     cnt_sh, si2, ones_v):
    c = lax.axis_index("c")
    s = lax.axis_index("s")
    rbase = s * ROWS_PER_TILE
    pltpu.sync_copy(zc.at[pl.ds(rbase, ROWS_PER_TILE)],
                    cnt_sh.at[pl.ds(rbase, ROWS_PER_TILE)])
    pltpu.sync_copy(oneh, ones_v)
    plsc.subcore_barrier()

    def body(sidx, cnt_out):
        def super_chunk(m, carry):
            q = s * N_SUPER + m
            pltpu.sync_copy(sidx.at[q], si2)
            for t in range(8):
                pltpu.sync_copy(ones_v, cnt_sh.at[si2.at[t]], add=True)
            return carry

        lax.fori_loop(0, N_SUPER, super_chunk, 0)
        plsc.subcore_barrier()
        pltpu.sync_copy(cnt_sh.at[pl.ds(rbase, ROWS_PER_TILE)],
                        cnt_out.at[pl.ds(rbase, ROWS_PER_TILE)])

    @pl.when(c == 0)
    def _():
        body(sidx_f, cnt_f)

    @pl.when(c == 1)
    def _():
        body(sidx_b, cnt_b)


def _sc_count_call(sidx_f, sidx_b, zc, oneh):
    mesh = plsc.VectorSubcoreMesh(core_axis_name="c", subcore_axis_name="s")
    fn = pl.kernel(
        _sc_count_kernel,
        out_type=(jax.ShapeDtypeStruct((N_PAD, 16), _F32),
                  jax.ShapeDtypeStruct((N_PAD, 16), _F32)),
        mesh=mesh,
        scratch_types=[
            pltpu.VMEM_SHARED((N_PAD, 16), _F32),
            pltpu.VMEM((8, 128), jnp.int32),
            pltpu.VMEM((128, 16), _F32),
        ],
    )
    return fn(sidx_f, sidx_b, zc, oneh)


def kernel(x, edge_index, edge_attr,
           cf_ep_w1, cf_ep_b1, cf_ep_w2, cf_ep_b2,
           cf_mm_w1, cf_mm_b1, cf_mm_w2, cf_mm_b2,
           cb_ep_w1, cb_ep_b1, cb_ep_w2, cb_ep_b2,
           cb_mm_w1, cb_mm_b1, cb_mm_w2, cb_mm_b2, alpha):
    src = edge_index[0]
    dst = edge_index[1]
    pad_e = E_PAD - N_EDGES
    zpad = jnp.zeros((pad_e,), jnp.int32)
    dump = jnp.full((pad_e,), N_NODES, jnp.int32)
    gidx_f = jnp.concatenate([src, zpad]).reshape(IDX_BLOCKS, 8, 128)
    sidx_f = jnp.concatenate([dst, dump]).reshape(IDX_BLOCKS, 8, 128)
    gidx_b = jnp.concatenate([dst, zpad]).reshape(IDX_BLOCKS, 8, 128)
    sidx_b = jnp.concatenate([src, dump]).reshape(IDX_BLOCKS, 8, 128)
    eat = jnp.concatenate([edge_attr, jnp.zeros((pad_e, ED), _F32)]).T

    wf, cfold_f, wb, cfold_b = _fold_call(
        cf_ep_w2, cf_ep_b2.reshape(1, H), cf_mm_w1, cf_mm_b1.reshape(1, H),
        cb_ep_w2, cb_ep_b2.reshape(1, H), cb_mm_w1, cb_mm_b1.reshape(1, H))
    xwf, xwb = _prep_call(x, cf_mm_w1, cb_mm_w1)
    xwf = xwf.reshape(2 * N_NODES, HH)
    xwb = xwb.reshape(2 * N_NODES, HH)
    gf = _gmat_call(eat, cf_ep_w1, cf_ep_b1.reshape(1, H), wf, cfold_f)
    gb = _gmat_call(eat, cb_ep_w1, cb_ep_b1.reshape(1, H), wb, cfold_b)
    gf = gf.reshape(2 * E_PAD, HH)
    gb = gb.reshape(2 * E_PAD, HH)

    zd = jnp.zeros((N_PAD, HH), _F32)
    zc = jnp.zeros((N_PAD, 16), _F32)
    oneh = jnp.zeros((128, 16), _F32).at[:, 0].set(1.0)

    sf = _sc_call(gidx_f, sidx_f, xwf, gf, zd)
    sb = _sc_call(gidx_b, sidx_b, xwb, gb, zd)
    cntf, cntb = _sc_count_call(sidx_f, sidx_b, zc, oneh)

    return _final_call(
        sf[0:N_NODES], sf[N_PAD:N_PAD + N_NODES], cntf[0:N_NODES],
        sb[0:N_NODES], sb[N_PAD:N_PAD + N_NODES], cntb[0:N_NODES],
        cf_mm_w2, cf_mm_b2.reshape(1, H), cb_mm_w2, cb_mm_b2.reshape(1, H),
        alpha.reshape(1, 1))


# speedup vs baseline: 1.0311x; 1.0204x over previous
import functools

import jax
import jax.numpy as jnp
from jax import lax
from jax.experimental import pallas as pl
from jax.experimental.pallas import tpu as pltpu
from jax.experimental.pallas import tpu_sc as plsc

N_NODES = 10000
N_EDGES = 160000
H = 256
ED = 4
HH = 128
ROWS_PER_TILE = 632
N_PAD = 16 * ROWS_PER_TILE
E_PER_TILE = 10240
E_PAD = 16 * E_PER_TILE
CHUNK = 64
SUPER = 1024
N_SUPER = E_PER_TILE // SUPER
IDX_BLOCKS = E_PAD // SUPER

_F32 = jnp.float32


def _fold_body(epw2f, epb2f, mmw1f, mmb1f, epw2b, epb2b, mmw1b, mmb1b,
               wf, cf, wb, cb):
    wf[...] = jnp.dot(epw2f[...], mmw1f[...], preferred_element_type=_F32)
    cf[...] = jnp.dot(epb2f[...], mmw1f[...], preferred_element_type=_F32) + mmb1f[...]
    wb[...] = jnp.dot(epw2b[...], mmw1b[...], preferred_element_type=_F32)
    cb[...] = jnp.dot(epb2b[...], mmw1b[...], preferred_element_type=_F32) + mmb1b[...]


def _prep_body(x, w1f, w1b, xwf, xwb):
    xf = jnp.dot(x[...], w1f[...], preferred_element_type=_F32)
    xwf[0] = xf[:, :HH]
    xwf[1] = xf[:, HH:]
    xb = jnp.dot(x[...], w1b[...], preferred_element_type=_F32)
    xwb[0] = xb[:, :HH]
    xwb[1] = xb[:, HH:]


def _gmat_body(eat, w1, b1, wfold, cfold, g):
    hid = jax.nn.relu(
        lax.dot_general(eat[...], w1[...], (((0,), (0,)), ((), ())),
                        preferred_element_type=_F32) + b1[...])
    gv = jnp.dot(hid, wfold[...], preferred_element_type=_F32) + cfold[...]
    g[0] = gv[:, :HH]
    g[1] = gv[:, HH:]


def _final_body(sfl, sfh, cntf, sbl, sbh, cntb, w2f, b2f, w2b, b2b, al, out):
    hf = (jnp.dot(sfl[...], w2f[:HH, :], preferred_element_type=_F32)
          + jnp.dot(sfh[...], w2f[HH:, :], preferred_element_type=_F32)
          + cntf[:, 0:1] * b2f[...])
    hb = (jnp.dot(sbl[...], w2b[:HH, :], preferred_element_type=_F32)
          + jnp.dot(sbh[...], w2b[HH:, :], preferred_element_type=_F32)
          + cntb[:, 0:1] * b2b[...])
    a = jax.nn.sigmoid(al[0, 0])
    out[...] = a * hf + (1.0 - a) * hb


def _fold_call(epw2f, epb2f, mmw1f, mmb1f, epw2b, epb2b, mmw1b, mmb1b):
    return pl.pallas_call(
        _fold_body,
        out_shape=[jax.ShapeDtypeStruct((H, H), _F32),
                   jax.ShapeDtypeStruct((1, H), _F32),
                   jax.ShapeDtypeStruct((H, H), _F32),
                   jax.ShapeDtypeStruct((1, H), _F32)],
    )(epw2f, epb2f, mmw1f, mmb1f, epw2b, epb2b, mmw1b, mmb1b)


def _prep_call(x, w1f, w1b):
    tn = 1000
    grid = N_NODES // tn
    return pl.pallas_call(
        _prep_body,
        grid=(grid,),
        in_specs=[pl.BlockSpec((tn, H), lambda i: (i, 0)),
                  pl.BlockSpec((H, H), lambda i: (0, 0)),
                  pl.BlockSpec((H, H), lambda i: (0, 0))],
        out_specs=[pl.BlockSpec((2, tn, HH), lambda i: (0, i, 0)),
                   pl.BlockSpec((2, tn, HH), lambda i: (0, i, 0))],
        out_shape=[jax.ShapeDtypeStruct((2, N_NODES, HH), _F32),
                   jax.ShapeDtypeStruct((2, N_NODES, HH), _F32)],
    )(x, w1f, w1b)


def _gmat_call(eat, w1, b1, wfold, cfold):
    te = 2048
    grid = E_PAD // te
    return pl.pallas_call(
        _gmat_body,
        grid=(grid,),
        in_specs=[pl.BlockSpec((ED, te), lambda i: (0, i)),
                  pl.BlockSpec((ED, H), lambda i: (0, 0)),
                  pl.BlockSpec((1, H), lambda i: (0, 0)),
                  pl.BlockSpec((H, H), lambda i: (0, 0)),
                  pl.BlockSpec((1, H), lambda i: (0, 0))],
        out_specs=pl.BlockSpec((2, te, HH), lambda i: (0, i, 0)),
        out_shape=jax.ShapeDtypeStruct((2, E_PAD, HH), _F32),
    )(eat, w1, b1, wfold, cfold)


def _final_call(sfl, sfh, cntf, sbl, sbh, cntb, w2f, b2f, w2b, b2b, al):
    tn = 1000
    grid = N_NODES // tn
    return pl.pallas_call(
        _final_body,
        grid=(grid,),
        in_specs=[pl.BlockSpec((tn, HH), lambda i: (i, 0)),
                  pl.BlockSpec((tn, HH), lambda i: (i, 0)),
                  pl.BlockSpec((tn, 16), lambda i: (i, 0)),
                  pl.BlockSpec((tn, HH), lambda i: (i, 0)),
                  pl.BlockSpec((tn, HH), lambda i: (i, 0)),
                  pl.BlockSpec((tn, 16), lambda i: (i, 0)),
                  pl.BlockSpec((H, H), lambda i: (0, 0)),
                  pl.BlockSpec((1, H), lambda i: (0, 0)),
                  pl.BlockSpec((H, H), lambda i: (0, 0)),
                  pl.BlockSpec((1, H), lambda i: (0, 0)),
                  pl.BlockSpec((1, 1), lambda i: (0, 0))],
        out_specs=pl.BlockSpec((tn, H), lambda i: (i, 0)),
        out_shape=jax.ShapeDtypeStruct((N_NODES, H), _F32),
    )(sfl, sfh, cntf, sbl, sbh, cntb, w2f, b2f, w2b, b2b, al)


def _sc_segment_kernel(gidx, sidx, xw, gmat, zd, s_out,
                       data_sh, gi2, si2, buf0, buf1, buf2, buf3,
                       sem_g, sem_g2, sem_b0, sem_b1, sem_b2, sem_b3):
    c = lax.axis_index("c")
    s = lax.axis_index("s")
    bufs = (buf0, buf1, buf2, buf3)
    bsem = (sem_b0, sem_b1, sem_b2, sem_b3)
    rbase = s * ROWS_PER_TILE
    pltpu.sync_copy(zd.at[pl.ds(rbase, ROWS_PER_TILE)],
                    data_sh.at[pl.ds(rbase, ROWS_PER_TILE)])
    plsc.subcore_barrier()

    goff = c * E_PAD
    xoff = c * N_NODES
    n_sub = SUPER // CHUNK

    def super_chunk(m, carry):
        q = s * N_SUPER + m
        pltpu.sync_copy(gidx.at[q], gi2)
        pltpu.sync_copy(sidx.at[q], si2)
        for r in range(8):
            for j in range(8):
                sl = pl.ds(j * 16, 16)
                gi2[r, sl] = gi2[r, sl] + xoff
        sbase = s * E_PER_TILE + m * SUPER
        ld = [None] * n_sub
        sc = [None] * n_sub
        gd = [None] * n_sub
        gsem = (sem_g, sem_g2)

        def gissue(u):
            return pltpu.async_copy(
                xw.at[gi2.at[u // 2, pl.ds((u % 2) * CHUNK, CHUNK)]],
                bufs[u % 4], gsem[u % 2], add=True)

        for k in range(3):
            ld[k] = pltpu.async_copy(
                gmat.at[pl.ds(goff + sbase + k * CHUNK, CHUNK)],
                bufs[k], bsem[k])
        ld[0].wait()
        gd[0] = gissue(0)
        ld[1].wait()
        gd[1] = gissue(1)
        for t in range(n_sub):
            buf = bufs[t % 4]
            gd[t].wait()
            if t + 2 < n_sub:
                ld[t + 2].wait()
                gd[t + 2] = gissue(t + 2)

            def edge(e, ecarry):
                for j in range(8):
                    sl = pl.ds(j * 16, 16)
                    buf[e, sl] = jnp.maximum(buf[e, sl], 0.0)
                return ecarry

            lax.fori_loop(0, CHUNK, edge, 0)
            sc[t] = pltpu.async_copy(
                buf, data_sh.at[si2.at[t // 2, pl.ds((t % 2) * CHUNK, CHUNK)]],
                bsem[t % 4], add=True)
            if t + 3 < n_sub:
                if t - 1 >= 0:
                    sc[t - 1].wait()
                ld[t + 3] = pltpu.async_copy(
                    gmat.at[pl.ds(goff + sbase + (t + 3) * CHUNK, CHUNK)],
                    bufs[(t + 3) % 4], bsem[(t + 3) % 4])
        for t in range(n_sub - 4, n_sub):
            sc[t].wait()
        return carry

    lax.fori_loop(0, N_SUPER, super_chunk, 0)
    plsc.subcore_barrier()
    pltpu.sync_copy(data_sh.at[pl.ds(rbase, ROWS_PER_TILE)],
                    s_out.at[pl.ds(c * N_PAD + rbase, ROWS_PER_TILE)])


def _sc_call(gidx, sidx, xw, gmat, zd):
    mesh = plsc.VectorSubcoreMesh(core_axis_name="c", subcore_axis_name="s")
    fn = pl.kernel(
        _sc_segment_kernel,
        out_type=jax.ShapeDtypeStruct((2 * N_PAD, HH), _F32),
        mesh=mesh,
        scratch_types=[
            pltpu.VMEM_SHARED((N_PAD, HH), _F32),
            pltpu.VMEM((8, 128), jnp.int32),
            pltpu.VMEM((8, 128), jnp.int32),
            pltpu.VMEM((CHUNK, 128), _F32),
            pltpu.VMEM((CHUNK, 128), _F32),
            pltpu.VMEM((CHUNK, 128), _F32),
            pltpu.VMEM((CHUNK, 128), _F32),
            pltpu.SemaphoreType.DMA,
            pltpu.SemaphoreType.DMA,
            pltpu.SemaphoreType.DMA,
            pltpu.SemaphoreType.DMA,
            pltpu.SemaphoreType.DMA,
            pltpu.SemaphoreType.DMA,
        ],
    )
    return fn(gidx, sidx, xw, gmat, zd)


def _sc_count_kernel(sidx_f, sidx_b, zc, oneh, cnt_f, cnt_b,
                     cnt_sh, si2, ones_v):
    c = lax.axis_index("c")
    s = lax.axis_index("s")
    rbase = s * ROWS_PER_TILE
    pltpu.sync_copy(zc.at[pl.ds(rbase, ROWS_PER_TILE)],
                    cnt_sh.at[pl.ds(rbase, ROWS_PER_TILE)])
    pltpu.sync_copy(oneh, ones_v)
    plsc.subcore_barrier()

    def body(sidx, cnt_out):
        def super_chunk(m, carry):
            q = s * N_SUPER + m
            pltpu.sync_copy(sidx.at[q], si2)
            for t in range(8):
                pltpu.sync_copy(ones_v, cnt_sh.at[si2.at[t]], add=True)
            return carry

        lax.fori_loop(0, N_SUPER, super_chunk, 0)
        plsc.subcore_barrier()
        pltpu.sync_copy(cnt_sh.at[pl.ds(rbase, ROWS_PER_TILE)],
                        cnt_out.at[pl.ds(rbase, ROWS_PER_TILE)])

    @pl.when(c == 0)
    def _():
        body(sidx_f, cnt_f)

    @pl.when(c == 1)
    def _():
        body(sidx_b, cnt_b)


def _sc_count_call(sidx_f, sidx_b, zc, oneh):
    mesh = plsc.VectorSubcoreMesh(core_axis_name="c", subcore_axis_name="s")
    fn = pl.kernel(
        _sc_count_kernel,
        out_type=(jax.ShapeDtypeStruct((N_PAD, 16), _F32),
                  jax.ShapeDtypeStruct((N_PAD, 16), _F32)),
        mesh=mesh,
        scratch_types=[
            pltpu.VMEM_SHARED((N_PAD, 16), _F32),
            pltpu.VMEM((8, 128), jnp.int32),
            pltpu.VMEM((128, 16), _F32),
        ],
    )
    return fn(sidx_f, sidx_b, zc, oneh)


def kernel(x, edge_index, edge_attr,
           cf_ep_w1, cf_ep_b1, cf_ep_w2, cf_ep_b2,
           cf_mm_w1, cf_mm_b1, cf_mm_w2, cf_mm_b2,
           cb_ep_w1, cb_ep_b1, cb_ep_w2, cb_ep_b2,
           cb_mm_w1, cb_mm_b1, cb_mm_w2, cb_mm_b2, alpha):
    src = edge_index[0]
    dst = edge_index[1]
    pad_e = E_PAD - N_EDGES
    zpad = jnp.zeros((pad_e,), jnp.int32)
    dump = jnp.full((pad_e,), N_NODES, jnp.int32)
    gidx_f = jnp.concatenate([src, zpad]).reshape(IDX_BLOCKS, 8, 128)
    sidx_f = jnp.concatenate([dst, dump]).reshape(IDX_BLOCKS, 8, 128)
    gidx_b = jnp.concatenate([dst, zpad]).reshape(IDX_BLOCKS, 8, 128)
    sidx_b = jnp.concatenate([src, dump]).reshape(IDX_BLOCKS, 8, 128)
    eat = jnp.concatenate([edge_attr, jnp.zeros((pad_e, ED), _F32)]).T

    wf, cfold_f, wb, cfold_b = _fold_call(
        cf_ep_w2, cf_ep_b2.reshape(1, H), cf_mm_w1, cf_mm_b1.reshape(1, H),
        cb_ep_w2, cb_ep_b2.reshape(1, H), cb_mm_w1, cb_mm_b1.reshape(1, H))
    xwf, xwb = _prep_call(x, cf_mm_w1, cb_mm_w1)
    xwf = xwf.reshape(2 * N_NODES, HH)
    xwb = xwb.reshape(2 * N_NODES, HH)
    gf = _gmat_call(eat, cf_ep_w1, cf_ep_b1.reshape(1, H), wf, cfold_f)
    gb = _gmat_call(eat, cb_ep_w1, cb_ep_b1.reshape(1, H), wb, cfold_b)
    gf = gf.reshape(2 * E_PAD, HH)
    gb = gb.reshape(2 * E_PAD, HH)

    zd = jnp.zeros((N_PAD, HH), _F32)
    zc = jnp.zeros((N_PAD, 16), _F32)
    oneh = jnp.zeros((128, 16), _F32).at[:, 0].set(1.0)

    sf = _sc_call(gidx_f, sidx_f, xwf, gf, zd)
    sb = _sc_call(gidx_b, sidx_b, xwb, gb, zd)
    cntf, cntb = _sc_count_call(sidx_f, sidx_b, zc, oneh)

    return _final_call(
        sf[0:N_NODES], sf[N_PAD:N_PAD + N_NODES], cntf[0:N_NODES],
        sb[0:N_NODES], sb[N_PAD:N_PAD + N_NODES], cntb[0:N_NODES],
        cf_mm_w2, cf_mm_b2.reshape(1, H), cb_mm_w2, cb_mm_b2.reshape(1, H),
        alpha.reshape(1, 1))
